# baseline (speedup 1.0000x reference)
import functools
import jax
import jax.numpy as jnp
from jax import lax
from jax.experimental import pallas as pl
from jax.experimental.pallas import tpu as pltpu

_N = 10000
_E = 320000
_H = 64
_HL = 32
_G = 64
_DE = 16
_BNS = 1.0 / (1.0 + 1e-5) ** 0.5

_RN = 2000
_EB = 4000


def _inproj_body(x_ref, w_ref, b_ref, wd_ref, bd_ref, ws_ref,
                 h_ref, a_ref, b2_ref):
    h = jnp.dot(x_ref[...], w_ref[...], preferred_element_type=jnp.float32)
    h = h + b_ref[...]
    h_ref[...] = h
    a_ref[...] = jnp.dot(h, wd_ref[...], preferred_element_type=jnp.float32) + bd_ref[...]
    b2_ref[...] = jnp.dot(h, ws_ref[...], preferred_element_type=jnp.float32)


def _edge_body(g_ref, ea_ref, we_ref, w2_ref, b2_ref, s2_ref, be2_ref, out_ref):
    t = g_ref[...] + jnp.dot(ea_ref[...], we_ref[...],
                             preferred_element_type=jnp.float32)
    r = jnp.maximum(t, 0.0)
    m2 = jnp.dot(r, w2_ref[...], preferred_element_type=jnp.float32) + b2_ref[...]
    out_ref[...] = jnp.maximum(m2, 0.0) * s2_ref[...] + be2_ref[...]


def _update_body(h_ref, ag_ref, w1h_ref, w1a_ref, b1_ref, w2_ref, b2_ref,
                 s2_ref, be2_ref, wd_ref, bd_ref, ws_ref,
                 h_out, a_out, b_out):
    h = h_ref[...]
    aggr = ag_ref[...]
    t = (jnp.dot(h, w1h_ref[...], preferred_element_type=jnp.float32)
         + jnp.dot(aggr, w1a_ref[...], preferred_element_type=jnp.float32)
         + b1_ref[...])
    r = jnp.maximum(t, 0.0)
    u = jnp.dot(r, w2_ref[...], preferred_element_type=jnp.float32) + b2_ref[...]
    u = jnp.maximum(u, 0.0) * s2_ref[...] + be2_ref[...]
    hn = h + u
    h_out[...] = hn
    a_out[...] = jnp.dot(hn, wd_ref[...], preferred_element_type=jnp.float32) + bd_ref[...]
    b_out[...] = jnp.dot(hn, ws_ref[...], preferred_element_type=jnp.float32)


def _final_body(h_ref, ag_ref, w1h_ref, w1a_ref, b1_ref, w2_ref, b2_ref,
                s2_ref, be2_ref, batch_ref, ow_ref, ob_ref, out_ref):
    h = h_ref[...]
    aggr = ag_ref[...]
    t = (jnp.dot(h, w1h_ref[...], preferred_element_type=jnp.float32)
         + jnp.dot(aggr, w1a_ref[...], preferred_element_type=jnp.float32)
         + b1_ref[...])
    r = jnp.maximum(t, 0.0)
    u = jnp.dot(r, w2_ref[...], preferred_element_type=jnp.float32) + b2_ref[...]
    u = jnp.maximum(u, 0.0) * s2_ref[...] + be2_ref[...]
    hn = h + u
    oh = (batch_ref[...] == lax.broadcasted_iota(jnp.int32, (1, _G), 1))
    ohf = oh.astype(jnp.float32)
    sums = lax.dot_general(ohf, hn, (((0,), (0,)), ((), ())),
                           preferred_element_type=jnp.float32)
    cnts = jnp.sum(ohf, axis=0)[:, None]
    mean = sums / jnp.maximum(cnts, 1.0)
    out_ref[...] = jnp.dot(mean, ow_ref[...],
                           preferred_element_type=jnp.float32) + ob_ref[...]


def _row_spec(bs, ncols):
    return pl.BlockSpec((bs, ncols), lambda i: (i, 0))


def _full_spec(shape):
    nd = len(shape)
    return pl.BlockSpec(shape, lambda i: (0,) * nd)


def kernel(x, edge_index, edge_attr, batch, lin_in_W, lin_in_b, msg_W1, msg_b1,
           msg_g1, msg_be1, msg_W2, msg_b2, msg_g2, msg_be2, upd_W1, upd_b1,
           upd_g1, upd_be1, upd_W2, upd_b2, upd_g2, upd_be2, out_W, out_b):
    L = msg_W1.shape[0]
    src = edge_index[0]
    dst = edge_index[1]

    ms1 = msg_g1 * _BNS
    ms2 = msg_g2 * _BNS
    us1 = upd_g1 * _BNS
    us2 = upd_g2 * _BNS
    mWd = msg_W1[:, :_H, :] * ms1[:, None, :]
    mWs = msg_W1[:, _H:2 * _H, :] * ms1[:, None, :]
    mWe = msg_W1[:, 2 * _H:, :] * ms1[:, None, :]
    mbd = msg_b1 * ms1 + msg_be1
    uWh = upd_W1[:, :_H, :] * us1[:, None, :]
    uWa = upd_W1[:, _H:, :] * us1[:, None, :]
    ub1 = upd_b1 * us1 + upd_be1

    f32 = jnp.float32

    inproj = pl.pallas_call(
        _inproj_body,
        grid=(_N // _RN,),
        in_specs=[_row_spec(_RN, 128), _full_spec((128, _H)), _full_spec((1, _H)),
                  _full_spec((_H, _HL)), _full_spec((1, _HL)), _full_spec((_H, _HL))],
        out_specs=[_row_spec(_RN, _H), _row_spec(_RN, _HL), _row_spec(_RN, _HL)],
        out_shape=[jax.ShapeDtypeStruct((_N, _H), f32),
                   jax.ShapeDtypeStruct((_N, _HL), f32),
                   jax.ShapeDtypeStruct((_N, _HL), f32)],
    )
    h, A, B = inproj(x, lin_in_W, lin_in_b.reshape(1, _H),
                     mWd[0], mbd[0].reshape(1, _HL), mWs[0])

    edge_mlp = pl.pallas_call(
        _edge_body,
        grid=(_E // _EB,),
        in_specs=[_row_spec(_EB, _HL), _row_spec(_EB, _DE),
                  _full_spec((_DE, _HL)), _full_spec((_HL, _H)),
                  _full_spec((1, _H)), _full_spec((1, _H)), _full_spec((1, _H))],
        out_specs=_row_spec(_EB, _H),
        out_shape=jax.ShapeDtypeStruct((_E, _H), f32),
    )

    update = pl.pallas_call(
        _update_body,
        grid=(_N // _RN,),
        in_specs=[_row_spec(_RN, _H), _row_spec(_RN, _H),
                  _full_spec((_H, _HL)), _full_spec((_H, _HL)), _full_spec((1, _HL)),
                  _full_spec((_HL, _H)), _full_spec((1, _H)), _full_spec((1, _H)),
                  _full_spec((1, _H)),
                  _full_spec((_H, _HL)), _full_spec((1, _HL)), _full_spec((_H, _HL))],
        out_specs=[_row_spec(_RN, _H), _row_spec(_RN, _HL), _row_spec(_RN, _HL)],
        out_shape=[jax.ShapeDtypeStruct((_N, _H), f32),
                   jax.ShapeDtypeStruct((_N, _HL), f32),
                   jax.ShapeDtypeStruct((_N, _HL), f32)],
    )

    final = pl.pallas_call(
        _final_body,
        grid=(1,),
        in_specs=[_full_spec((_N, _H)), _full_spec((_N, _H)),
                  _full_spec((_H, _HL)), _full_spec((_H, _HL)), _full_spec((1, _HL)),
                  _full_spec((_HL, _H)), _full_spec((1, _H)), _full_spec((1, _H)),
                  _full_spec((1, _H)),
                  _full_spec((_N, 1)), _full_spec((_H, 1)), _full_spec((1, 1))],
        out_specs=_full_spec((_G, 1)),
        out_shape=jax.ShapeDtypeStruct((_G, 1), f32),
    )

    for l in range(L):
        g = jnp.take(A, dst, axis=0) + jnp.take(B, src, axis=0)
        eout = edge_mlp(g, edge_attr, mWe[l], msg_W2[l],
                        msg_b2[l].reshape(1, _H), ms2[l].reshape(1, _H),
                        msg_be2[l].reshape(1, _H))
        aggr = jax.ops.segment_sum(eout, dst, num_segments=_N)
        if l + 1 < L:
            h, A, B = update(h, aggr, uWh[l], uWa[l], ub1[l].reshape(1, _HL),
                             upd_W2[l], upd_b2[l].reshape(1, _H),
                             us2[l].reshape(1, _H), upd_be2[l].reshape(1, _H),
                             mWd[l + 1], mbd[l + 1].reshape(1, _HL), mWs[l + 1])
        else:
            out = final(h, aggr, uWh[l], uWa[l], ub1[l].reshape(1, _HL),
                        upd_W2[l], upd_b2[l].reshape(1, _H),
                        us2[l].reshape(1, _H), upd_be2[l].reshape(1, _H),
                        batch.reshape(_N, 1), out_W, out_b.reshape(1, 1))
    return out.reshape(-1)


# speedup vs baseline: 2.3525x; 2.3525x over previous
import functools
import jax
import jax.numpy as jnp
from jax import lax
from jax.experimental import pallas as pl
from jax.experimental.pallas import tpu as pltpu
from jax.experimental.pallas import tpu_sc as plsc

_N = 10000
_E = 320000
_H = 64
_HL = 32
_G = 64
_DE = 16
_BNS = 1.0 / (1.0 + 1e-5) ** 0.5

_RN = 2000
_EB = 4000

_NC = 2
_NS = 16
_NW = _NC * _NS
_PW = _E // _NW
_GB = 128
_NB = _PW // _GB
_TL = _PW - _NB * _GB
_NRS = _N // _NS


def _inproj_body(x_ref, w_ref, b_ref, wd_ref, bd_ref, ws_ref,
                 h_ref, a_ref, b2_ref):
    h = jnp.dot(x_ref[...], w_ref[...], preferred_element_type=jnp.float32)
    h = h + b_ref[...]
    h_ref[...] = h
    a_ref[...] = jnp.dot(h, wd_ref[...], preferred_element_type=jnp.float32) + bd_ref[...]
    b2_ref[...] = jnp.dot(h, ws_ref[...], preferred_element_type=jnp.float32)


def _edge_body(ga_ref, gb_ref, ea_ref, we_ref, w2_ref, b2_ref, s2_ref,
               be2_ref, out_ref):
    t = ga_ref[...] + gb_ref[...] + jnp.dot(
        ea_ref[...], we_ref[...], preferred_element_type=jnp.float32)
    r = jnp.maximum(t, 0.0)
    m2 = jnp.dot(r, w2_ref[...], preferred_element_type=jnp.float32) + b2_ref[...]
    out_ref[...] = jnp.maximum(m2, 0.0) * s2_ref[...] + be2_ref[...]


def _update_body(h_ref, p0_ref, p1_ref, w1h_ref, w1a_ref, b1_ref, w2_ref,
                 b2_ref, s2_ref, be2_ref, wd_ref, bd_ref, ws_ref,
                 h_out, a_out, b_out):
    h = h_ref[...]
    aggr = p0_ref[...] + p1_ref[...]
    t = (jnp.dot(h, w1h_ref[...], preferred_element_type=jnp.float32)
         + jnp.dot(aggr, w1a_ref[...], preferred_element_type=jnp.float32)
         + b1_ref[...])
    r = jnp.maximum(t, 0.0)
    u = jnp.dot(r, w2_ref[...], preferred_element_type=jnp.float32) + b2_ref[...]
    u = jnp.maximum(u, 0.0) * s2_ref[...] + be2_ref[...]
    hn = h + u
    h_out[...] = hn
    a_out[...] = jnp.dot(hn, wd_ref[...], preferred_element_type=jnp.float32) + bd_ref[...]
    b_out[...] = jnp.dot(hn, ws_ref[...], preferred_element_type=jnp.float32)


def _final_body(h_ref, p0_ref, p1_ref, w1h_ref, w1a_ref, b1_ref, w2_ref,
                b2_ref, s2_ref, be2_ref, batch_ref, ow_ref, ob_ref, out_ref):
    h = h_ref[...]
    aggr = p0_ref[...] + p1_ref[...]
    t = (jnp.dot(h, w1h_ref[...], preferred_element_type=jnp.float32)
         + jnp.dot(aggr, w1a_ref[...], preferred_element_type=jnp.float32)
         + b1_ref[...])
    r = jnp.maximum(t, 0.0)
    u = jnp.dot(r, w2_ref[...], preferred_element_type=jnp.float32) + b2_ref[...]
    u = jnp.maximum(u, 0.0) * s2_ref[...] + be2_ref[...]
    hn = h + u
    oh = (batch_ref[...] == lax.broadcasted_iota(jnp.int32, (1, _G), 1))
    ohf = oh.astype(jnp.float32)
    sums = lax.dot_general(ohf, hn, (((0,), (0,)), ((), ())),
                           preferred_element_type=jnp.float32)
    cnts = jnp.sum(ohf, axis=0)[:, None]
    mean = sums / jnp.maximum(cnts, 1.0)
    out_ref[...] = jnp.dot(mean, ow_ref[...],
                           preferred_element_type=jnp.float32) + ob_ref[...]


def _row_spec(bs, ncols):
    return pl.BlockSpec((bs, ncols), lambda i: (i, 0))


def _full_spec(shape):
    nd = len(shape)
    return pl.BlockSpec(shape, lambda i: (0,) * nd)


def _gather_body(a_hbm, b_hbm, src_hbm, dst_hbm, ga_hbm, gb_hbm,
                 idx_v, ra_v, rb_v, sem):
    c = lax.axis_index("c")
    s = lax.axis_index("s")
    base = (s * _NC + c) * _PW

    def blk(j, carry):
        off = base + j * _GB
        pltpu.sync_copy(dst_hbm.at[pl.ds(off, _GB)], idx_v.at[0])
        pltpu.sync_copy(src_hbm.at[pl.ds(off, _GB)], idx_v.at[1])
        cp_a = pltpu.async_copy(a_hbm.at[idx_v.at[0]], ra_v, sem)
        cp_b = pltpu.async_copy(b_hbm.at[idx_v.at[1]], rb_v, sem)
        cp_a.wait()
        cp_b.wait()
        pltpu.sync_copy(ra_v, ga_hbm.at[pl.ds(off, _GB)])
        pltpu.sync_copy(rb_v, gb_hbm.at[pl.ds(off, _GB)])
        return carry

    lax.fori_loop(0, _NB, blk, 0)
    off = base + _NB * _GB
    pltpu.sync_copy(dst_hbm.at[pl.ds(off, _TL)], idx_v.at[0, pl.ds(0, _TL)])
    pltpu.sync_copy(src_hbm.at[pl.ds(off, _TL)], idx_v.at[1, pl.ds(0, _TL)])
    cp_a = pltpu.async_copy(a_hbm.at[idx_v.at[0, pl.ds(0, _TL)]],
                            ra_v.at[pl.ds(0, _TL)], sem)
    cp_b = pltpu.async_copy(b_hbm.at[idx_v.at[1, pl.ds(0, _TL)]],
                            rb_v.at[pl.ds(0, _TL)], sem)
    cp_a.wait()
    cp_b.wait()
    pltpu.sync_copy(ra_v.at[pl.ds(0, _TL)], ga_hbm.at[pl.ds(off, _TL)])
    pltpu.sync_copy(rb_v.at[pl.ds(0, _TL)], gb_hbm.at[pl.ds(off, _TL)])


def _scatter_body(eo_hbm, dst_hbm, z_hbm, out_hbm,
                  idx_v, idx_t, rows_v, rows_t, acc_sh):
    c = lax.axis_index("c")
    s = lax.axis_index("s")
    base = (s * _NC + c) * _PW
    r0 = s * _NRS
    pltpu.sync_copy(z_hbm.at[pl.ds(r0, _NRS)], acc_sh.at[pl.ds(r0, _NRS)])
    plsc.subcore_barrier()

    def blk(j, carry):
        off = base + j * _GB
        pltpu.sync_copy(dst_hbm.at[pl.ds(off, _GB)], idx_v.at[0])
        pltpu.sync_copy(eo_hbm.at[pl.ds(off, _GB)], rows_v)
        pltpu.sync_copy(rows_v, acc_sh.at[idx_v.at[0]], add=True)
        return carry

    lax.fori_loop(0, _NB, blk, 0)
    off = base + _NB * _GB
    pltpu.sync_copy(dst_hbm.at[pl.ds(off, _TL)], idx_t.at[0])
    pltpu.sync_copy(eo_hbm.at[pl.ds(off, _TL)], rows_t)
    pltpu.sync_copy(rows_t, acc_sh.at[idx_t.at[0]], add=True)
    plsc.subcore_barrier()
    pltpu.sync_copy(acc_sh.at[pl.ds(r0, _NRS)],
                    out_hbm.at[pl.ds(c * _N + r0, _NRS)])


def kernel(x, edge_index, edge_attr, batch, lin_in_W, lin_in_b, msg_W1, msg_b1,
           msg_g1, msg_be1, msg_W2, msg_b2, msg_g2, msg_be2, upd_W1, upd_b1,
           upd_g1, upd_be1, upd_W2, upd_b2, upd_g2, upd_be2, out_W, out_b):
    L = msg_W1.shape[0]
    src = edge_index[0]
    dst = edge_index[1]

    ms1 = msg_g1 * _BNS
    ms2 = msg_g2 * _BNS
    us1 = upd_g1 * _BNS
    us2 = upd_g2 * _BNS
    mWd = msg_W1[:, :_H, :] * ms1[:, None, :]
    mWs = msg_W1[:, _H:2 * _H, :] * ms1[:, None, :]
    mWe = msg_W1[:, 2 * _H:, :] * ms1[:, None, :]
    mbd = msg_b1 * ms1 + msg_be1
    uWh = upd_W1[:, :_H, :] * us1[:, None, :]
    uWa = upd_W1[:, _H:, :] * us1[:, None, :]
    ub1 = upd_b1 * us1 + upd_be1

    f32 = jnp.float32
    zeros_n = jnp.zeros((_N, _H), dtype=f32)

    inproj = pl.pallas_call(
        _inproj_body,
        grid=(_N // _RN,),
        in_specs=[_row_spec(_RN, 128), _full_spec((128, _H)), _full_spec((1, _H)),
                  _full_spec((_H, _HL)), _full_spec((1, _HL)), _full_spec((_H, _HL))],
        out_specs=[_row_spec(_RN, _H), _row_spec(_RN, _HL), _row_spec(_RN, _HL)],
        out_shape=[jax.ShapeDtypeStruct((_N, _H), f32),
                   jax.ShapeDtypeStruct((_N, _HL), f32),
                   jax.ShapeDtypeStruct((_N, _HL), f32)],
    )
    h, A, B = inproj(x, lin_in_W, lin_in_b.reshape(1, _H),
                     mWd[0], mbd[0].reshape(1, _HL), mWs[0])

    edge_mlp = pl.pallas_call(
        _edge_body,
        grid=(_E // _EB,),
        in_specs=[_row_spec(_EB, _HL), _row_spec(_EB, _HL), _row_spec(_EB, _DE),
                  _full_spec((_DE, _HL)), _full_spec((_HL, _H)),
                  _full_spec((1, _H)), _full_spec((1, _H)), _full_spec((1, _H))],
        out_specs=_row_spec(_EB, _H),
        out_shape=jax.ShapeDtypeStruct((_E, _H), f32),
    )

    update = pl.pallas_call(
        _update_body,
        grid=(_N // _RN,),
        in_specs=[_row_spec(_RN, _H), _row_spec(_RN, _H), _row_spec(_RN, _H),
                  _full_spec((_H, _HL)), _full_spec((_H, _HL)), _full_spec((1, _HL)),
                  _full_spec((_HL, _H)), _full_spec((1, _H)), _full_spec((1, _H)),
                  _full_spec((1, _H)),
                  _full_spec((_H, _HL)), _full_spec((1, _HL)), _full_spec((_H, _HL))],
        out_specs=[_row_spec(_RN, _H), _row_spec(_RN, _HL), _row_spec(_RN, _HL)],
        out_shape=[jax.ShapeDtypeStruct((_N, _H), f32),
                   jax.ShapeDtypeStruct((_N, _HL), f32),
                   jax.ShapeDtypeStruct((_N, _HL), f32)],
    )

    final = pl.pallas_call(
        _final_body,
        grid=(1,),
        in_specs=[_full_spec((_N, _H)), _full_spec((_N, _H)), _full_spec((_N, _H)),
                  _full_spec((_H, _HL)), _full_spec((_H, _HL)), _full_spec((1, _HL)),
                  _full_spec((_HL, _H)), _full_spec((1, _H)), _full_spec((1, _H)),
                  _full_spec((1, _H)),
                  _full_spec((_N, 1)), _full_spec((_H, 1)), _full_spec((1, 1))],
        out_specs=_full_spec((_G, 1)),
        out_shape=jax.ShapeDtypeStruct((_G, 1), f32),
    )

    mesh = plsc.VectorSubcoreMesh(core_axis_name="c", subcore_axis_name="s")

    gather_sc = pl.kernel(
        _gather_body,
        out_type=[jax.ShapeDtypeStruct((_E, _HL), f32),
                  jax.ShapeDtypeStruct((_E, _HL), f32)],
        scratch_types=[pltpu.VMEM((2, _GB), jnp.int32),
                       pltpu.VMEM((_GB, _HL), f32),
                       pltpu.VMEM((_GB, _HL), f32),
                       pltpu.SemaphoreType.DMA],
        mesh=mesh,
        compiler_params=pltpu.CompilerParams(use_tc_tiling_on_sc=False),
    )

    scatter_sc = pl.kernel(
        _scatter_body,
        out_type=jax.ShapeDtypeStruct((_NC * _N, _H), f32),
        scratch_types=[pltpu.VMEM((1, _GB), jnp.int32),
                       pltpu.VMEM((1, _TL), jnp.int32),
                       pltpu.VMEM((_GB, _H), f32),
                       pltpu.VMEM((_TL, _H), f32),
                       pltpu.VMEM_SHARED((_N, _H), f32)],
        mesh=mesh,
        compiler_params=pltpu.CompilerParams(use_tc_tiling_on_sc=False),
    )

    for l in range(L):
        gA, gB = gather_sc(A, B, src, dst)
        eout = edge_mlp(gA, gB, edge_attr, mWe[l], msg_W2[l],
                        msg_b2[l].reshape(1, _H), ms2[l].reshape(1, _H),
                        msg_be2[l].reshape(1, _H))
        parts = scatter_sc(eout, dst, zeros_n)
        p0 = parts[:_N]
        p1 = parts[_N:]
        if l + 1 < L:
            h, A, B = update(h, p0, p1, uWh[l], uWa[l], ub1[l].reshape(1, _HL),
                             upd_W2[l], upd_b2[l].reshape(1, _H),
                             us2[l].reshape(1, _H), upd_be2[l].reshape(1, _H),
                             mWd[l + 1], mbd[l + 1].reshape(1, _HL), mWs[l + 1])
        else:
            out = final(h, p0, p1, uWh[l], uWa[l], ub1[l].reshape(1, _HL),
                        upd_W2[l], upd_b2[l].reshape(1, _H),
                        us2[l].reshape(1, _H), upd_be2[l].reshape(1, _H),
                        batch.reshape(_N, 1), out_W, out_b.reshape(1, 1))
    return out.reshape(-1)


# speedup vs baseline: 2.9936x; 1.2725x over previous
import functools
import jax
import jax.numpy as jnp
from jax import lax
from jax.experimental import pallas as pl
from jax.experimental.pallas import tpu as pltpu
from jax.experimental.pallas import tpu_sc as plsc

_N = 10000
_E = 320000
_H = 64
_HL = 32
_G = 64
_DE = 16
_BNS = 1.0 / (1.0 + 1e-5) ** 0.5

_RN = 2000
_EB = 4000

_NC = 2
_NS = 16
_NW = _NC * _NS
_GB = 128
_EBLK = _E // _GB
_WB = _EBLK // _NW
_XTRA = _EBLK - _WB * _NW
_K = 6
_NGRP = _WB // _K
_NRS = _N // _NS


def _inproj_body(x_ref, w_ref, b_ref, wd_ref, bd_ref, ws_ref,
                 h_ref, a_ref, b2_ref):
    h = jnp.dot(x_ref[...], w_ref[...], preferred_element_type=jnp.float32)
    h = h + b_ref[...]
    h_ref[...] = h
    a_ref[...] = jnp.dot(h, wd_ref[...], preferred_element_type=jnp.float32) + bd_ref[...]
    b2_ref[...] = jnp.dot(h, ws_ref[...], preferred_element_type=jnp.float32)


def _edge_body(ga_ref, gb_ref, ea_ref, we_ref, w2_ref, b2_ref, s2_ref,
               be2_ref, out_ref):
    t = ga_ref[...] + gb_ref[...] + jnp.dot(
        ea_ref[...], we_ref[...], preferred_element_type=jnp.float32)
    r = jnp.maximum(t, 0.0)
    m2 = jnp.dot(r, w2_ref[...], preferred_element_type=jnp.float32) + b2_ref[...]
    out_ref[...] = jnp.maximum(m2, 0.0) * s2_ref[...] + be2_ref[...]


def _update_body(h_ref, p0_ref, p1_ref, w1h_ref, w1a_ref, b1_ref, w2_ref,
                 b2_ref, s2_ref, be2_ref, wd_ref, bd_ref, ws_ref,
                 h_out, a_out, b_out):
    h = h_ref[...]
    aggr = p0_ref[...] + p1_ref[...]
    t = (jnp.dot(h, w1h_ref[...], preferred_element_type=jnp.float32)
         + jnp.dot(aggr, w1a_ref[...], preferred_element_type=jnp.float32)
         + b1_ref[...])
    r = jnp.maximum(t, 0.0)
    u = jnp.dot(r, w2_ref[...], preferred_element_type=jnp.float32) + b2_ref[...]
    u = jnp.maximum(u, 0.0) * s2_ref[...] + be2_ref[...]
    hn = h + u
    h_out[...] = hn
    a_out[...] = jnp.dot(hn, wd_ref[...], preferred_element_type=jnp.float32) + bd_ref[...]
    b_out[...] = jnp.dot(hn, ws_ref[...], preferred_element_type=jnp.float32)


def _final_body(h_ref, p0_ref, p1_ref, w1h_ref, w1a_ref, b1_ref, w2_ref,
                b2_ref, s2_ref, be2_ref, batch_ref, ow_ref, ob_ref, out_ref):
    h = h_ref[...]
    aggr = p0_ref[...] + p1_ref[...]
    t = (jnp.dot(h, w1h_ref[...], preferred_element_type=jnp.float32)
         + jnp.dot(aggr, w1a_ref[...], preferred_element_type=jnp.float32)
         + b1_ref[...])
    r = jnp.maximum(t, 0.0)
    u = jnp.dot(r, w2_ref[...], preferred_element_type=jnp.float32) + b2_ref[...]
    u = jnp.maximum(u, 0.0) * s2_ref[...] + be2_ref[...]
    hn = h + u
    oh = (batch_ref[...] == lax.broadcasted_iota(jnp.int32, (1, _G), 1))
    ohf = oh.astype(jnp.float32)
    sums = lax.dot_general(ohf, hn, (((0,), (0,)), ((), ())),
                           preferred_element_type=jnp.float32)
    cnts = jnp.sum(ohf, axis=0)[:, None]
    mean = sums / jnp.maximum(cnts, 1.0)
    out_ref[...] = jnp.dot(mean, ow_ref[...],
                           preferred_element_type=jnp.float32) + ob_ref[...]


def _row_spec(bs, ncols):
    return pl.BlockSpec((bs, ncols), lambda i: (i, 0))


def _full_spec(shape):
    nd = len(shape)
    return pl.BlockSpec(shape, lambda i: (0,) * nd)


def _gather_body(a_hbm, b_hbm, src2_hbm, dst2_hbm, ga_hbm, gb_hbm,
                 idxd, idxs, bufa, bufb, xidx, sem_g, sem_w):
    c = lax.axis_index("c")
    s = lax.axis_index("s")
    w = s * _NC + c
    row0 = w * _WB
    pltpu.sync_copy(dst2_hbm.at[pl.ds(row0, _WB)], idxd)
    pltpu.sync_copy(src2_hbm.at[pl.ds(row0, _WB)], idxs)

    def grp(g, carry):
        k0 = g * _K
        cps = []
        for b in range(_K):
            cps.append(pltpu.async_copy(a_hbm.at[idxd.at[k0 + b]],
                                        bufa.at[b], sem_g))
            cps.append(pltpu.async_copy(b_hbm.at[idxs.at[k0 + b]],
                                        bufb.at[b], sem_g))
        for cp in cps:
            cp.wait()
        wbs = []
        for b in range(_K):
            off = (row0 + k0 + b) * _GB
            wbs.append(pltpu.async_copy(bufa.at[b],
                                        ga_hbm.at[pl.ds(off, _GB)], sem_w))
            wbs.append(pltpu.async_copy(bufb.at[b],
                                        gb_hbm.at[pl.ds(off, _GB)], sem_w))
        for cp in wbs:
            cp.wait()
        return carry

    lax.fori_loop(0, _NGRP, grp, 0)

    @pl.when(w < _XTRA)
    def _():
        j = _NW * _WB + w
        pltpu.sync_copy(dst2_hbm.at[pl.ds(j, 1)], xidx.at[pl.ds(0, 1)])
        pltpu.sync_copy(src2_hbm.at[pl.ds(j, 1)], xidx.at[pl.ds(1, 1)])
        cp_a = pltpu.async_copy(a_hbm.at[xidx.at[0]], bufa.at[0], sem_g)
        cp_b = pltpu.async_copy(b_hbm.at[xidx.at[1]], bufb.at[0], sem_g)
        cp_a.wait()
        cp_b.wait()
        off = j * _GB
        pltpu.sync_copy(bufa.at[0], ga_hbm.at[pl.ds(off, _GB)])
        pltpu.sync_copy(bufb.at[0], gb_hbm.at[pl.ds(off, _GB)])


def _scatter_body(eo_hbm, dst2_hbm, z_hbm, out_hbm,
                  idxd, xidx, rows_v, acc_sh, sem_l):
    c = lax.axis_index("c")
    s = lax.axis_index("s")
    w = s * _NC + c
    row0 = w * _WB
    r0 = s * _NRS
    pltpu.sync_copy(z_hbm.at[pl.ds(r0, _NRS)], acc_sh.at[pl.ds(r0, _NRS)])
    pltpu.sync_copy(dst2_hbm.at[pl.ds(row0, _WB)], idxd)
    plsc.subcore_barrier()

    def grp(g, carry):
        k0 = g * _K
        cps = []
        for b in range(_K):
            off = (row0 + k0 + b) * _GB
            cps.append(pltpu.async_copy(eo_hbm.at[pl.ds(off, _GB)],
                                        rows_v.at[b], sem_l))
        for b in range(_K):
            cps[b].wait()
            pltpu.sync_copy(rows_v.at[b], acc_sh.at[idxd.at[k0 + b]], add=True)
        return carry

    lax.fori_loop(0, _NGRP, grp, 0)

    @pl.when(w < _XTRA)
    def _():
        j = _NW * _WB + w
        pltpu.sync_copy(dst2_hbm.at[pl.ds(j, 1)], xidx.at[pl.ds(0, 1)])
        pltpu.sync_copy(eo_hbm.at[pl.ds(j * _GB, _GB)], rows_v.at[0])
        pltpu.sync_copy(rows_v.at[0], acc_sh.at[xidx.at[0]], add=True)

    plsc.subcore_barrier()
    pltpu.sync_copy(acc_sh.at[pl.ds(r0, _NRS)],
                    out_hbm.at[pl.ds(c * _N + r0, _NRS)])


def kernel(x, edge_index, edge_attr, batch, lin_in_W, lin_in_b, msg_W1, msg_b1,
           msg_g1, msg_be1, msg_W2, msg_b2, msg_g2, msg_be2, upd_W1, upd_b1,
           upd_g1, upd_be1, upd_W2, upd_b2, upd_g2, upd_be2, out_W, out_b):
    L = msg_W1.shape[0]
    src = edge_index[0]
    dst = edge_index[1]

    ms1 = msg_g1 * _BNS
    ms2 = msg_g2 * _BNS
    us1 = upd_g1 * _BNS
    us2 = upd_g2 * _BNS
    mWd = msg_W1[:, :_H, :] * ms1[:, None, :]
    mWs = msg_W1[:, _H:2 * _H, :] * ms1[:, None, :]
    mWe = msg_W1[:, 2 * _H:, :] * ms1[:, None, :]
    mbd = msg_b1 * ms1 + msg_be1
    uWh = upd_W1[:, :_H, :] * us1[:, None, :]
    uWa = upd_W1[:, _H:, :] * us1[:, None, :]
    ub1 = upd_b1 * us1 + upd_be1

    f32 = jnp.float32
    zeros_n = jnp.zeros((_N, _H), dtype=f32)

    inproj = pl.pallas_call(
        _inproj_body,
        grid=(_N // _RN,),
        in_specs=[_row_spec(_RN, 128), _full_spec((128, _H)), _full_spec((1, _H)),
                  _full_spec((_H, _HL)), _full_spec((1, _HL)), _full_spec((_H, _HL))],
        out_specs=[_row_spec(_RN, _H), _row_spec(_RN, _HL), _row_spec(_RN, _HL)],
        out_shape=[jax.ShapeDtypeStruct((_N, _H), f32),
                   jax.ShapeDtypeStruct((_N, _HL), f32),
                   jax.ShapeDtypeStruct((_N, _HL), f32)],
    )
    h, A, B = inproj(x, lin_in_W, lin_in_b.reshape(1, _H),
                     mWd[0], mbd[0].reshape(1, _HL), mWs[0])

    edge_mlp = pl.pallas_call(
        _edge_body,
        grid=(_E // _EB,),
        in_specs=[_row_spec(_EB, _HL), _row_spec(_EB, _HL), _row_spec(_EB, _DE),
                  _full_spec((_DE, _HL)), _full_spec((_HL, _H)),
                  _full_spec((1, _H)), _full_spec((1, _H)), _full_spec((1, _H))],
        out_specs=_row_spec(_EB, _H),
        out_shape=jax.ShapeDtypeStruct((_E, _H), f32),
    )

    update = pl.pallas_call(
        _update_body,
        grid=(_N // _RN,),
        in_specs=[_row_spec(_RN, _H), _row_spec(_RN, _H), _row_spec(_RN, _H),
                  _full_spec((_H, _HL)), _full_spec((_H, _HL)), _full_spec((1, _HL)),
                  _full_spec((_HL, _H)), _full_spec((1, _H)), _full_spec((1, _H)),
                  _full_spec((1, _H)),
                  _full_spec((_H, _HL)), _full_spec((1, _HL)), _full_spec((_H, _HL))],
        out_specs=[_row_spec(_RN, _H), _row_spec(_RN, _HL), _row_spec(_RN, _HL)],
        out_shape=[jax.ShapeDtypeStruct((_N, _H), f32),
                   jax.ShapeDtypeStruct((_N, _HL), f32),
                   jax.ShapeDtypeStruct((_N, _HL), f32)],
    )

    final = pl.pallas_call(
        _final_body,
        grid=(1,),
        in_specs=[_full_spec((_N, _H)), _full_spec((_N, _H)), _full_spec((_N, _H)),
                  _full_spec((_H, _HL)), _full_spec((_H, _HL)), _full_spec((1, _HL)),
                  _full_spec((_HL, _H)), _full_spec((1, _H)), _full_spec((1, _H)),
                  _full_spec((1, _H)),
                  _full_spec((_N, 1)), _full_spec((_H, 1)), _full_spec((1, 1))],
        out_specs=_full_spec((_G, 1)),
        out_shape=jax.ShapeDtypeStruct((_G, 1), f32),
    )

    mesh = plsc.VectorSubcoreMesh(core_axis_name="c", subcore_axis_name="s")

    gather_sc = pl.kernel(
        _gather_body,
        out_type=[jax.ShapeDtypeStruct((_E, _HL), f32),
                  jax.ShapeDtypeStruct((_E, _HL), f32)],
        scratch_types=[pltpu.VMEM((_WB, _GB), jnp.int32),
                       pltpu.VMEM((_WB, _GB), jnp.int32),
                       pltpu.VMEM((_K, _GB, _HL), f32),
                       pltpu.VMEM((_K, _GB, _HL), f32),
                       pltpu.VMEM((2, _GB), jnp.int32),
                       pltpu.SemaphoreType.DMA,
                       pltpu.SemaphoreType.DMA],
        mesh=mesh,
        compiler_params=pltpu.CompilerParams(use_tc_tiling_on_sc=False),
    )

    scatter_sc = pl.kernel(
        _scatter_body,
        out_type=jax.ShapeDtypeStruct((_NC * _N, _H), f32),
        scratch_types=[pltpu.VMEM((_WB, _GB), jnp.int32),
                       pltpu.VMEM((2, _GB), jnp.int32),
                       pltpu.VMEM((_K, _GB, _H), f32),
                       pltpu.VMEM_SHARED((_N, _H), f32),
                       pltpu.SemaphoreType.DMA],
        mesh=mesh,
        compiler_params=pltpu.CompilerParams(use_tc_tiling_on_sc=False),
    )

    src2 = src.reshape(_EBLK, _GB)
    dst2 = dst.reshape(_EBLK, _GB)

    for l in range(L):
        gA, gB = gather_sc(A, B, src2, dst2)
        eout = edge_mlp(gA, gB, edge_attr, mWe[l], msg_W2[l],
                        msg_b2[l].reshape(1, _H), ms2[l].reshape(1, _H),
                        msg_be2[l].reshape(1, _H))
        parts = scatter_sc(eout, dst2, zeros_n)
        p0 = parts[:_N]
        p1 = parts[_N:]
        if l + 1 < L:
            h, A, B = update(h, p0, p1, uWh[l], uWa[l], ub1[l].reshape(1, _HL),
                             upd_W2[l], upd_b2[l].reshape(1, _H),
                             us2[l].reshape(1, _H), upd_be2[l].reshape(1, _H),
                             mWd[l + 1], mbd[l + 1].reshape(1, _HL), mWs[l + 1])
        else:
            out = final(h, p0, p1, uWh[l], uWa[l], ub1[l].reshape(1, _HL),
                        upd_W2[l], upd_b2[l].reshape(1, _H),
                        us2[l].reshape(1, _H), upd_be2[l].reshape(1, _H),
                        batch.reshape(_N, 1), out_W, out_b.reshape(1, 1))
    return out.reshape(-1)


# speedup vs baseline: 3.0416x; 1.0161x over previous
import functools
import jax
import jax.numpy as jnp
from jax import lax
from jax.experimental import pallas as pl
from jax.experimental.pallas import tpu as pltpu
from jax.experimental.pallas import tpu_sc as plsc

_N = 10000
_E = 320000
_H = 64
_HL = 32
_G = 64
_DE = 16
_BNS = 1.0 / (1.0 + 1e-5) ** 0.5

_RN = 2000
_EB = 4000

_NC = 2
_NS = 16
_NW = _NC * _NS
_GB = 128
_EBLK = _E // _GB
_NP = 2
_PB = _EBLK // _NP
_PE = _E // _NP
_WB = _PB // _NW
_XTRA = _PB - _WB * _NW
_K = 13
_NGRP = _WB // _K
_KS = 3
_NGS = _WB // _KS
_NRS = _N // _NS


def _inproj_body(x_ref, w_ref, b_ref, wd_ref, bd_ref, ws_ref,
                 h_ref, a_ref, b2_ref):
    h = jnp.dot(x_ref[...], w_ref[...], preferred_element_type=jnp.float32)
    h = h + b_ref[...]
    h_ref[...] = h
    a_ref[...] = jnp.dot(h, wd_ref[...], preferred_element_type=jnp.float32) + bd_ref[...]
    b2_ref[...] = jnp.dot(h, ws_ref[...], preferred_element_type=jnp.float32)


def _edge_body(ga_ref, gb_ref, ea_ref, we_ref, w2_ref, b2_ref, s2_ref,
               be2_ref, out_ref):
    t = ga_ref[...] + gb_ref[...] + jnp.dot(
        ea_ref[...], we_ref[...], preferred_element_type=jnp.float32)
    r = jnp.maximum(t, 0.0)
    m2 = jnp.dot(r, w2_ref[...], preferred_element_type=jnp.float32) + b2_ref[...]
    out_ref[...] = jnp.maximum(m2, 0.0) * s2_ref[...] + be2_ref[...]


def _update_body(h_ref, p00_ref, p01_ref, p10_ref, p11_ref,
                 w1h_ref, w1a_ref, b1_ref, w2_ref,
                 b2_ref, s2_ref, be2_ref, wd_ref, bd_ref, ws_ref,
                 h_out, a_out, b_out):
    h = h_ref[...]
    aggr = ((p00_ref[...] + p01_ref[...])
            + (p10_ref[...] + p11_ref[...]))
    t = (jnp.dot(h, w1h_ref[...], preferred_element_type=jnp.float32)
         + jnp.dot(aggr, w1a_ref[...], preferred_element_type=jnp.float32)
         + b1_ref[...])
    r = jnp.maximum(t, 0.0)
    u = jnp.dot(r, w2_ref[...], preferred_element_type=jnp.float32) + b2_ref[...]
    u = jnp.maximum(u, 0.0) * s2_ref[...] + be2_ref[...]
    hn = h + u
    h_out[...] = hn
    a_out[...] = jnp.dot(hn, wd_ref[...], preferred_element_type=jnp.float32) + bd_ref[...]
    b_out[...] = jnp.dot(hn, ws_ref[...], preferred_element_type=jnp.float32)


def _final_body(h_ref, p00_ref, p01_ref, p10_ref, p11_ref,
                w1h_ref, w1a_ref, b1_ref, w2_ref,
                b2_ref, s2_ref, be2_ref, batch_ref, ow_ref, ob_ref, out_ref):
    h = h_ref[...]
    aggr = ((p00_ref[...] + p01_ref[...])
            + (p10_ref[...] + p11_ref[...]))
    t = (jnp.dot(h, w1h_ref[...], preferred_element_type=jnp.float32)
         + jnp.dot(aggr, w1a_ref[...], preferred_element_type=jnp.float32)
         + b1_ref[...])
    r = jnp.maximum(t, 0.0)
    u = jnp.dot(r, w2_ref[...], preferred_element_type=jnp.float32) + b2_ref[...]
    u = jnp.maximum(u, 0.0) * s2_ref[...] + be2_ref[...]
    hn = h + u
    oh = (batch_ref[...] == lax.broadcasted_iota(jnp.int32, (1, _G), 1))
    ohf = oh.astype(jnp.float32)
    sums = lax.dot_general(ohf, hn, (((0,), (0,)), ((), ())),
                           preferred_element_type=jnp.float32)
    cnts = jnp.sum(ohf, axis=0)[:, None]
    mean = sums / jnp.maximum(cnts, 1.0)
    out_ref[...] = jnp.dot(mean, ow_ref[...],
                           preferred_element_type=jnp.float32) + ob_ref[...]


def _row_spec(bs, ncols):
    return pl.BlockSpec((bs, ncols), lambda i: (i, 0))


def _full_spec(shape):
    nd = len(shape)
    return pl.BlockSpec(shape, lambda i: (0,) * nd)


def _make_gather_body(pbase):
    def body(a_hbm, b_hbm, src2_hbm, dst2_hbm, ga_hbm, gb_hbm,
             idxd, idxs, bufa, bufb, xidx, sem_g, sem_w):
        c = lax.axis_index("c")
        s = lax.axis_index("s")
        w = s * _NC + c
        lrow0 = w * _WB
        pltpu.sync_copy(dst2_hbm.at[pl.ds(pbase + lrow0, _WB)], idxd)
        pltpu.sync_copy(src2_hbm.at[pl.ds(pbase + lrow0, _WB)], idxs)

        def grp(g, carry):
            k0 = g * _K
            cps = []
            for b in range(_K):
                cps.append(pltpu.async_copy(a_hbm.at[idxd.at[k0 + b]],
                                            bufa.at[b], sem_g))
                cps.append(pltpu.async_copy(b_hbm.at[idxs.at[k0 + b]],
                                            bufb.at[b], sem_g))
            for cp in cps:
                cp.wait()
            wbs = []
            for b in range(_K):
                off = (lrow0 + k0 + b) * _GB
                wbs.append(pltpu.async_copy(bufa.at[b],
                                            ga_hbm.at[pl.ds(off, _GB)], sem_w))
                wbs.append(pltpu.async_copy(bufb.at[b],
                                            gb_hbm.at[pl.ds(off, _GB)], sem_w))
            for cp in wbs:
                cp.wait()
            return carry

        lax.fori_loop(0, _NGRP, grp, 0)

        @pl.when(w < _XTRA)
        def _():
            lj = _NW * _WB + w
            pltpu.sync_copy(dst2_hbm.at[pl.ds(pbase + lj, 1)],
                            xidx.at[pl.ds(0, 1)])
            pltpu.sync_copy(src2_hbm.at[pl.ds(pbase + lj, 1)],
                            xidx.at[pl.ds(1, 1)])
            cp_a = pltpu.async_copy(a_hbm.at[xidx.at[0]], bufa.at[0], sem_g)
            cp_b = pltpu.async_copy(b_hbm.at[xidx.at[1]], bufb.at[0], sem_g)
            cp_a.wait()
            cp_b.wait()
            off = lj * _GB
            pltpu.sync_copy(bufa.at[0], ga_hbm.at[pl.ds(off, _GB)])
            pltpu.sync_copy(bufb.at[0], gb_hbm.at[pl.ds(off, _GB)])

    return body


def _make_scatter_body(pbase):
    def body(eo_hbm, dst2_hbm, z_hbm, out_hbm,
             idxd, xidx, rows_v, acc_sh, sem_l):
        c = lax.axis_index("c")
        s = lax.axis_index("s")
        w = s * _NC + c
        lrow0 = w * _WB
        r0 = s * _NRS
        pltpu.sync_copy(z_hbm.at[pl.ds(r0, _NRS)], acc_sh.at[pl.ds(r0, _NRS)])
        pltpu.sync_copy(dst2_hbm.at[pl.ds(pbase + lrow0, _WB)], idxd)
        plsc.subcore_barrier()

        def grp(g, carry):
            k0 = g * _KS
            cps = []
            for b in range(_KS):
                off = (lrow0 + k0 + b) * _GB
                cps.append(pltpu.async_copy(eo_hbm.at[pl.ds(off, _GB)],
                                            rows_v.at[b], sem_l))
            for b in range(_KS):
                cps[b].wait()
                pltpu.sync_copy(rows_v.at[b], acc_sh.at[idxd.at[k0 + b]],
                                add=True)
            return carry

        lax.fori_loop(0, _NGS, grp, 0)

        @pl.when(w < _XTRA)
        def _():
            lj = _NW * _WB + w
            pltpu.sync_copy(dst2_hbm.at[pl.ds(pbase + lj, 1)],
                            xidx.at[pl.ds(0, 1)])
            pltpu.sync_copy(eo_hbm.at[pl.ds(lj * _GB, _GB)], rows_v.at[0])
            pltpu.sync_copy(rows_v.at[0], acc_sh.at[xidx.at[0]], add=True)

        plsc.subcore_barrier()
        pltpu.sync_copy(acc_sh.at[pl.ds(r0, _NRS)],
                        out_hbm.at[pl.ds(c * _N + r0, _NRS)])

    return body


def kernel(x, edge_index, edge_attr, batch, lin_in_W, lin_in_b, msg_W1, msg_b1,
           msg_g1, msg_be1, msg_W2, msg_b2, msg_g2, msg_be2, upd_W1, upd_b1,
           upd_g1, upd_be1, upd_W2, upd_b2, upd_g2, upd_be2, out_W, out_b):
    L = msg_W1.shape[0]
    src = edge_index[0]
    dst = edge_index[1]

    ms1 = msg_g1 * _BNS
    ms2 = msg_g2 * _BNS
    us1 = upd_g1 * _BNS
    us2 = upd_g2 * _BNS
    mWd = msg_W1[:, :_H, :] * ms1[:, None, :]
    mWs = msg_W1[:, _H:2 * _H, :] * ms1[:, None, :]
    mWe = msg_W1[:, 2 * _H:, :] * ms1[:, None, :]
    mbd = msg_b1 * ms1 + msg_be1
    uWh = upd_W1[:, :_H, :] * us1[:, None, :]
    uWa = upd_W1[:, _H:, :] * us1[:, None, :]
    ub1 = upd_b1 * us1 + upd_be1

    f32 = jnp.float32
    zeros_n = jnp.zeros((_N, _H), dtype=f32)

    inproj = pl.pallas_call(
        _inproj_body,
        grid=(_N // _RN,),
        in_specs=[_row_spec(_RN, 128), _full_spec((128, _H)), _full_spec((1, _H)),
                  _full_spec((_H, _HL)), _full_spec((1, _HL)), _full_spec((_H, _HL))],
        out_specs=[_row_spec(_RN, _H), _row_spec(_RN, _HL), _row_spec(_RN, _HL)],
        out_shape=[jax.ShapeDtypeStruct((_N, _H), f32),
                   jax.ShapeDtypeStruct((_N, _HL), f32),
                   jax.ShapeDtypeStruct((_N, _HL), f32)],
    )
    h, A, B = inproj(x, lin_in_W, lin_in_b.reshape(1, _H),
                     mWd[0], mbd[0].reshape(1, _HL), mWs[0])

    edge_mlp = pl.pallas_call(
        _edge_body,
        grid=(_PE // _EB,),
        in_specs=[_row_spec(_EB, _HL), _row_spec(_EB, _HL), _row_spec(_EB, _DE),
                  _full_spec((_DE, _HL)), _full_spec((_HL, _H)),
                  _full_spec((1, _H)), _full_spec((1, _H)), _full_spec((1, _H))],
        out_specs=_row_spec(_EB, _H),
        out_shape=jax.ShapeDtypeStruct((_PE, _H), f32),
    )

    update = pl.pallas_call(
        _update_body,
        grid=(_N // _RN,),
        in_specs=[_row_spec(_RN, _H),
                  _row_spec(_RN, _H), _row_spec(_RN, _H),
                  _row_spec(_RN, _H), _row_spec(_RN, _H),
                  _full_spec((_H, _HL)), _full_spec((_H, _HL)), _full_spec((1, _HL)),
                  _full_spec((_HL, _H)), _full_spec((1, _H)), _full_spec((1, _H)),
                  _full_spec((1, _H)),
                  _full_spec((_H, _HL)), _full_spec((1, _HL)), _full_spec((_H, _HL))],
        out_specs=[_row_spec(_RN, _H), _row_spec(_RN, _HL), _row_spec(_RN, _HL)],
        out_shape=[jax.ShapeDtypeStruct((_N, _H), f32),
                   jax.ShapeDtypeStruct((_N, _HL), f32),
                   jax.ShapeDtypeStruct((_N, _HL), f32)],
    )

    final = pl.pallas_call(
        _final_body,
        grid=(1,),
        in_specs=[_full_spec((_N, _H)),
                  _full_spec((_N, _H)), _full_spec((_N, _H)),
                  _full_spec((_N, _H)), _full_spec((_N, _H)),
                  _full_spec((_H, _HL)), _full_spec((_H, _HL)), _full_spec((1, _HL)),
                  _full_spec((_HL, _H)), _full_spec((1, _H)), _full_spec((1, _H)),
                  _full_spec((1, _H)),
                  _full_spec((_N, 1)), _full_spec((_H, 1)), _full_spec((1, 1))],
        out_specs=_full_spec((_G, 1)),
        out_shape=jax.ShapeDtypeStruct((_G, 1), f32),
    )

    mesh = plsc.VectorSubcoreMesh(core_axis_name="c", subcore_axis_name="s")
    scp = pltpu.CompilerParams(use_tc_tiling_on_sc=False)

    gather_scratch = [pltpu.VMEM((_WB, _GB), jnp.int32),
                      pltpu.VMEM((_WB, _GB), jnp.int32),
                      pltpu.VMEM((_K, _GB, _HL), f32),
                      pltpu.VMEM((_K, _GB, _HL), f32),
                      pltpu.VMEM((2, _GB), jnp.int32),
                      pltpu.SemaphoreType.DMA,
                      pltpu.SemaphoreType.DMA]
    scatter_scratch = [pltpu.VMEM((_WB, _GB), jnp.int32),
                       pltpu.VMEM((2, _GB), jnp.int32),
                       pltpu.VMEM((_KS, _GB, _H), f32),
                       pltpu.VMEM_SHARED((_N, _H), f32),
                       pltpu.SemaphoreType.DMA]

    gather_p = []
    scatter_p = []
    for p in range(_NP):
        gather_p.append(pl.kernel(
            _make_gather_body(p * _PB),
            out_type=[jax.ShapeDtypeStruct((_PE, _HL), f32),
                      jax.ShapeDtypeStruct((_PE, _HL), f32)],
            scratch_types=gather_scratch,
            mesh=mesh,
            compiler_params=scp,
        ))
        scatter_p.append(pl.kernel(
            _make_scatter_body(p * _PB),
            out_type=jax.ShapeDtypeStruct((_NC * _N, _H), f32),
            scratch_types=scatter_scratch,
            mesh=mesh,
            compiler_params=scp,
        ))

    src2 = src.reshape(_EBLK, _GB)
    dst2 = dst.reshape(_EBLK, _GB)
    ea_p = [edge_attr[p * _PE:(p + 1) * _PE] for p in range(_NP)]

    for l in range(L):
        parts = []
        for p in range(_NP):
            gA, gB = gather_p[p](A, B, src2, dst2)
            eout = edge_mlp(gA, gB, ea_p[p], mWe[l], msg_W2[l],
                            msg_b2[l].reshape(1, _H), ms2[l].reshape(1, _H),
                            msg_be2[l].reshape(1, _H))
            parts.append(scatter_p[p](eout, dst2, zeros_n))
        p00 = parts[0][:_N]
        p01 = parts[0][_N:]
        p10 = parts[1][:_N]
        p11 = parts[1][_N:]
        if l + 1 < L:
            h, A, B = update(h, p00, p01, p10, p11,
                             uWh[l], uWa[l], ub1[l].reshape(1, _HL),
                             upd_W2[l], upd_b2[l].reshape(1, _H),
                             us2[l].reshape(1, _H), upd_be2[l].reshape(1, _H),
                             mWd[l + 1], mbd[l + 1].reshape(1, _HL), mWs[l + 1])
        else:
            out = final(h, p00, p01, p10, p11,
                        uWh[l], uWa[l], ub1[l].reshape(1, _HL),
                        upd_W2[l], upd_b2[l].reshape(1, _H),
                        us2[l].reshape(1, _H), upd_be2[l].reshape(1, _H),
                        batch.reshape(_N, 1), out_W, out_b.reshape(1, 1))
    return out.reshape(-1)
